# baseline (speedup 1.0000x reference)
import functools

import jax
import jax.numpy as jnp
from jax import lax
from jax.experimental import pallas as pl
from jax.experimental.pallas import tpu as pltpu
from jax.experimental.pallas import tpu_sc as plsc

HIDDEN = 1024
EPS = 1e-12

_NUM_CORES = 2
_NUM_SUBCORES = 16
_NW = _NUM_CORES * _NUM_SUBCORES
_LANES = 16
_V = 3


def _combos_body(t_ref, g_ref, b_ref, o_ref):
    t = t_ref[...]
    mean = jnp.mean(t, axis=-1, keepdims=True)
    cent = t - mean
    var = jnp.mean(cent * cent, axis=-1, keepdims=True)
    normed = cent * lax.rsqrt(var + EPS) * g_ref[...] + b_ref[...]
    rows = []
    for a in range(_V):
        for b in range(_V):
            for c in range(_V):
                rows.append(
                    jnp.concatenate(
                        [normed[a : a + 1], normed[b : b + 1], normed[c : c + 1]],
                        axis=-1,
                    )
                )
    o_ref[...] = jnp.concatenate(rows, axis=0)


def _make_combos(table, gamma, beta):
    v, h = table.shape
    return pl.pallas_call(
        _combos_body,
        out_shape=jax.ShapeDtypeStruct((v * v * v, 3 * h), jnp.float32),
    )(table, gamma.reshape(1, h), beta.reshape(1, h))


@functools.lru_cache(maxsize=None)
def _make_scatter(n_tokens, h):
    bpw = n_tokens // _NW
    nsg = bpw // 48
    tail = bpw - nsg * 48
    assert tail == 16
    ntrip = nsg * 16 + 5
    ch = 3 * h
    mesh = plsc.VectorSubcoreMesh(core_axis_name="c", subcore_axis_name="s")

    @functools.partial(
        pl.kernel,
        mesh=mesh,
        out_type=jax.ShapeDtypeStruct((n_tokens * h,), jnp.float32),
        scratch_types=[
            pltpu.VMEM((bpw,), jnp.int32),
            pltpu.VMEM((27 * ch,), jnp.float32),
            pltpu.SemaphoreType.DMA,
        ],
    )
    def k(combos_hbm, idx_hbm, out_hbm, idx_v, comb_v, sem):
        wid = lax.axis_index("s") * _NUM_CORES + lax.axis_index("c")
        base = wid * bpw
        pltpu.sync_copy(combos_hbm, comb_v)
        pltpu.sync_copy(idx_hbm.at[wid], idx_v)

        def supergroup(g, carry):
            goff = pl.multiple_of(g * 48, 8)
            i0 = idx_v[pl.ds(goff, _LANES)]
            i1 = idx_v[pl.ds(goff + 16, _LANES)]
            i2 = idx_v[pl.ds(goff + 32, _LANES)]
            lanes = [i0[t] for t in range(16)] + [i1[t] for t in range(16)] + [
                i2[t] for t in range(16)
            ]
            gbase = (base + g * 48) * h
            for t in range(16):
                combo = lanes[3 * t] * 9 + lanes[3 * t + 1] * 3 + lanes[3 * t + 2]
                src = pl.multiple_of(combo * ch, 8)
                dst = pl.multiple_of(gbase + t * ch, 8)
                pltpu.async_copy(
                    comb_v.at[pl.ds(src, ch)], out_hbm.at[pl.ds(dst, ch)], sem
                )
            return carry

        lax.fori_loop(0, nsg, supergroup, 0)

        toff = nsg * 48
        it = idx_v[pl.ds(toff, _LANES)]
        tbase = (base + toff) * h
        for t in range(5):
            combo = it[3 * t] * 9 + it[3 * t + 1] * 3 + it[3 * t + 2]
            src = pl.multiple_of(combo * ch, 8)
            dst = pl.multiple_of(tbase + t * ch, 8)
            pltpu.async_copy(
                comb_v.at[pl.ds(src, ch)], out_hbm.at[pl.ds(dst, ch)], sem
            )
        src1 = pl.multiple_of(it[15] * 13 * ch, 8)
        dst1 = pl.multiple_of(tbase + 15 * h, 8)
        pltpu.async_copy(
            comb_v.at[pl.ds(src1, h)], out_hbm.at[pl.ds(dst1, h)], sem
        )

        def drain(i, carry):
            pltpu.make_async_copy(
                comb_v.at[pl.ds(0, ch)],
                out_hbm.at[pl.ds(base * h, ch)],
                sem,
            ).wait()
            return carry

        lax.fori_loop(0, ntrip, drain, 0)
        pltpu.make_async_copy(
            comb_v.at[pl.ds(0, h)], out_hbm.at[pl.ds(base * h, h)], sem
        ).wait()

    return k


def kernel(sentiment_input, table, gamma, beta):
    v, h = table.shape
    idx = sentiment_input.reshape(-1).astype(jnp.int32)
    n_tokens = idx.shape[0]
    combos = _make_combos(table, gamma, beta)
    scatter = _make_scatter(n_tokens, h)
    out = scatter(combos.reshape(-1), idx.reshape(_NW, n_tokens // _NW))
    return out.reshape(sentiment_input.shape + (h,))
